# baseline (speedup 1.0000x reference)
import functools

import jax
import jax.numpy as jnp
from jax import lax
from jax.experimental import pallas as pl
from jax.experimental.pallas import tpu as pltpu
from jax.experimental.pallas import tpu_sc as plsc

_R = 4
_UB = 16


def _stage_kernel(tid_ref, ep_ref, sel4_ref, sel_buf, rep_buf, gsem, wsem):
    cp = pltpu.make_async_copy(ep_ref.at[:, tid_ref[0]], sel_buf, gsem)
    cp.start()
    cp.wait()
    rep_buf[...] = jnp.broadcast_to(sel_buf[...][:, None], rep_buf.shape)
    ocp = pltpu.make_async_copy(rep_buf, sel4_ref, wsem)
    ocp.start()
    ocp.wait()


def _tc_stage(e_p, task_id):
    nL, nT, P, D = e_p.shape
    tid = jnp.asarray(task_id, jnp.int32).reshape((1,))
    return pl.pallas_call(
        _stage_kernel,
        grid_spec=pltpu.PrefetchScalarGridSpec(
            num_scalar_prefetch=1,
            grid=(1,),
            in_specs=[pl.BlockSpec(memory_space=pl.ANY)],
            out_specs=pl.BlockSpec(memory_space=pl.ANY),
            scratch_shapes=[
                pltpu.VMEM((nL, P, D), e_p.dtype),
                pltpu.VMEM((nL, _R, P, D), e_p.dtype),
                pltpu.SemaphoreType.DMA,
                pltpu.SemaphoreType.DMA,
            ],
        ),
        out_shape=jax.ShapeDtypeStruct((nL, _R, P, D), e_p.dtype),
    )(tid, e_p)


def kernel(x_query, vis_mark, e_p, task_id):
    del vis_mark
    B = x_query.shape[0]
    nL, nT, P, D = e_p.shape
    sel4 = _tc_stage(e_p, task_id)

    info = plsc.get_sparse_core_info()
    NC, NS = info.num_cores, info.num_subcores
    NW = NC * NS
    upl = B // _UB
    nunits = nL * upl
    upw = nunits // NW
    gpu = _UB // _R
    mesh = plsc.VectorSubcoreMesh(core_axis_name="c", subcore_axis_name="s")

    @functools.partial(
        pl.kernel,
        out_type=jax.ShapeDtypeStruct((nL, B, P, D), jnp.float32),
        mesh=mesh,
        scratch_types=[
            pltpu.VMEM((_R, P, D), jnp.float32),
            pltpu.SemaphoreType.DMA,
            pltpu.SemaphoreType.DMA,
        ],
    )
    def sc_fn(sel4_hbm, out_hbm, big, gsem, wsem):
        wid = lax.axis_index("s") * NC + lax.axis_index("c")
        u0 = wid * upw
        ls = [(u0 + k) // upl for k in range(upw)]
        b0s = [((u0 + k) % upl) * _UB for k in range(upw)]

        def write(k, i):
            return pltpu.make_async_copy(
                big, out_hbm.at[ls[k], pl.ds(b0s[k] + i * _R, _R)], wsem
            )

        for k in range(upw):
            if k > 0:
                for i in range(gpu):
                    write(k - 1, i).wait()
            fcp = pltpu.make_async_copy(sel4_hbm.at[ls[k]], big, gsem)
            fcp.start()
            fcp.wait()
            for i in range(gpu):
                write(k, i).start()
        for i in range(gpu):
            write(upw - 1, i).wait()

    return sc_fn(sel4)


# speedup vs baseline: 1.0069x; 1.0069x over previous
import functools

import jax
import jax.numpy as jnp
from jax import lax
from jax.experimental import pallas as pl
from jax.experimental.pallas import tpu as pltpu
from jax.experimental.pallas import tpu_sc as plsc

_R = 4


def _stage_kernel(tid_ref, ep_ref, sel4_ref, sel_buf, rep_buf, gsem, wsem):
    cp = pltpu.make_async_copy(ep_ref.at[:, tid_ref[0]], sel_buf, gsem)
    cp.start()
    cp.wait()
    rep_buf[...] = jnp.broadcast_to(sel_buf[...][:, None], rep_buf.shape)
    ocp = pltpu.make_async_copy(rep_buf, sel4_ref, wsem)
    ocp.start()
    ocp.wait()


def _tc_stage(e_p, task_id):
    nL, nT, P, D = e_p.shape
    tid = jnp.asarray(task_id, jnp.int32).reshape((1,))
    return pl.pallas_call(
        _stage_kernel,
        grid_spec=pltpu.PrefetchScalarGridSpec(
            num_scalar_prefetch=1,
            grid=(1,),
            in_specs=[pl.BlockSpec(memory_space=pl.ANY)],
            out_specs=pl.BlockSpec(memory_space=pl.ANY),
            scratch_shapes=[
                pltpu.VMEM((nL, P, D), e_p.dtype),
                pltpu.VMEM((nL, _R, P, D), e_p.dtype),
                pltpu.SemaphoreType.DMA,
                pltpu.SemaphoreType.DMA,
            ],
        ),
        out_shape=jax.ShapeDtypeStruct((nL, _R, P, D), e_p.dtype),
    )(tid, e_p)


def kernel(x_query, vis_mark, e_p, task_id):
    del vis_mark
    B = x_query.shape[0]
    nL, nT, P, D = e_p.shape
    sel4 = _tc_stage(e_p, task_id)

    info = plsc.get_sparse_core_info()
    NC, NS = info.num_cores, info.num_subcores
    NW = NC * NS
    wpl = 2
    nbw = B // wpl
    ngrp = nbw // _R
    mesh = plsc.VectorSubcoreMesh(core_axis_name="c", subcore_axis_name="s")

    @functools.partial(
        pl.kernel,
        out_type=jax.ShapeDtypeStruct((nL, B, P, D), jnp.float32),
        mesh=mesh,
        scratch_types=[
            pltpu.VMEM((_R, P, D), jnp.float32),
            pltpu.SemaphoreType.DMA,
            pltpu.SemaphoreType.DMA,
        ],
    )
    def sc_fn(sel4_hbm, out_hbm, big, gsem, wsem):
        wid = lax.axis_index("s") * NC + lax.axis_index("c")

        @pl.when(wid < nL * wpl)
        def _():
            l = wid // wpl
            b0 = (wid % wpl) * nbw
            fcp = pltpu.make_async_copy(sel4_hbm.at[l], big, gsem)
            fcp.start()
            fcp.wait()

            def write(i):
                return pltpu.make_async_copy(
                    big, out_hbm.at[l, pl.ds(b0 + i * _R, _R)], wsem
                )

            for i in range(ngrp):
                write(i).start()
            for i in range(ngrp):
                write(i).wait()

    return sc_fn(sel4)
